# baseline (speedup 1.0000x reference)
import functools

import jax
import jax.numpy as jnp
from jax import lax
from jax.experimental import pallas as pl
from jax.experimental.pallas import tpu as pltpu
from jax.experimental.pallas import tpu_sc as plsc

_L = 16


def _rsqrt16(x):
    i = lax.bitcast_convert_type(x, jnp.int32)
    i = jnp.int32(0x5F3759DF) - lax.shift_right_logical(i, 1)
    y = lax.bitcast_convert_type(i, jnp.float32)
    for _ in range(3):
        y = y * (jnp.float32(1.5) - jnp.float32(0.5) * x * y * y)
    return y


def _fused_sc(ids_flat, table, pos_table, tok_w, tok_b, pos_w, pos_b, B, S):
    N = ids_flat.shape[0]
    V, D = table.shape
    nj = D // _L
    info = plsc.get_sparse_core_info()
    nw = info.num_cores * info.num_subcores
    s_per_w = S // nw
    ch = 32
    n_sc = s_per_w // ch
    T = n_sc * B
    mesh = plsc.VectorSubcoreMesh(core_axis_name="c", subcore_axis_name="s")

    @functools.partial(
        pl.kernel,
        mesh=mesh,
        out_type=jax.ShapeDtypeStruct((N, D), jnp.float32),
        compiler_params=pltpu.CompilerParams(needs_layout_passes=False),
        scratch_types=[
            pltpu.VMEM((ch,), jnp.int32),
            pltpu.VMEM((ch,), jnp.int32),
            pltpu.VMEM((ch, D), jnp.float32),
            pltpu.VMEM((ch, D), jnp.float32),
            pltpu.VMEM((ch, D), jnp.float32),
            pltpu.VMEM((D,), jnp.float32),
            pltpu.VMEM((D,), jnp.float32),
            pltpu.VMEM((D,), jnp.float32),
            pltpu.VMEM((D,), jnp.float32),
            pltpu.SemaphoreType.DMA,
            pltpu.SemaphoreType.DMA,
            pltpu.SemaphoreType.DMA,
            pltpu.SemaphoreType.DMA,
        ],
    )
    def fused(ids_hbm, table_hbm, pos_hbm, tw_hbm, tb_hbm, pw_hbm, pb_hbm,
              out_hbm, idx0, idx1, rows0, rows1, add_v, tw_v, pw_v, pb_v,
              tb_v, gsem0, gsem1, ssem0, ssem1):
        wid = lax.axis_index("s") * info.num_cores + lax.axis_index("c")
        s0 = wid * s_per_w
        idx = (idx0, idx1)
        rows = (rows0, rows1)
        gsem = (gsem0, gsem1)
        ssem = (ssem0, ssem1)

        pltpu.sync_copy(tw_hbm, tw_v)
        pltpu.sync_copy(pw_hbm, pw_v)
        pltpu.sync_copy(pb_hbm, pb_v)
        pltpu.sync_copy(tb_hbm, tb_v)
        for j in range(nj):
            d = pl.ds(j * _L, _L)
            pb_v[d] = pb_v[d] + tb_v[d]

        def ln_rows(buf, scale_v, out_fn):
            def row_body(r, carry):
                acc = [jnp.zeros((_L,), jnp.float32) for _ in range(4)]
                accq = [jnp.zeros((_L,), jnp.float32) for _ in range(4)]
                for j in range(nj):
                    x = buf[r, pl.ds(j * _L, _L)]
                    acc[j % 4] = acc[j % 4] + x
                    accq[j % 4] = accq[j % 4] + x * x
                s = jnp.sum((acc[0] + acc[1]) + (acc[2] + acc[3]))
                q = jnp.sum((accq[0] + accq[1]) + (accq[2] + accq[3]))
                mu = s * jnp.float32(1.0 / 1024.0)
                var = q * jnp.float32(1.0 / 1024.0) - mu * mu
                a = _rsqrt16(jnp.full((_L,), var + jnp.float32(1e-5)))
                mu_v = jnp.full((_L,), mu)
                for j in range(nj):
                    d = pl.ds(j * _L, _L)
                    y = (buf[r, d] - mu_v) * scale_v[d] * a
                    out_fn(r, y, d)
                return carry
            lax.fori_loop(0, ch, row_body, 0)

        def step(k, j):
            t = 2 * k + j
            sc = t // B
            b = lax.rem(t, B)
            oj = 1 - j
            @pl.when(t >= 1)
            def _():
                pltpu.make_async_copy(
                    rows[oj], out_hbm.at[pl.ds(0, ch)], ssem[oj]).wait()
            @pl.when(t < T - 1)
            def _():
                t1 = t + 1
                off1 = lax.rem(t1, B) * S + s0 + (t1 // B) * ch
                pltpu.sync_copy(ids_hbm.at[pl.ds(off1, ch)], idx[oj])
                pltpu.make_async_copy(
                    table_hbm.at[idx[oj]], rows[oj], gsem[oj]).start()
            pltpu.make_async_copy(
                table_hbm.at[idx[j]], rows[j], gsem[j]).wait()
            @pl.when(b == 0)
            def _():
                pltpu.sync_copy(
                    pos_hbm.at[pl.ds(s0 + sc * ch, ch)], add_v)
                def store_add(r, y, d):
                    add_v[r, d] = y + pb_v[d]
                ln_rows(add_v, pw_v, store_add)
            def store_tok(r, y, d):
                rows[j][r, d] = y + add_v[r, d]
            ln_rows(rows[j], tw_v, store_tok)
            off = b * S + s0 + sc * ch
            pltpu.make_async_copy(
                rows[j], out_hbm.at[pl.ds(off, ch)], ssem[j]).start()

        pltpu.sync_copy(ids_hbm.at[pl.ds(s0, ch)], idx[0])
        pltpu.make_async_copy(table_hbm.at[idx[0]], rows[0], gsem[0]).start()

        def outer(k, carry):
            step(k, 0)
            step(k, 1)
            return carry
        lax.fori_loop(0, T // 2, outer, 0)

        pltpu.make_async_copy(rows[1], out_hbm.at[pl.ds(0, ch)], ssem[1]).wait()

    return fused(ids_flat, table, pos_table, tok_w, tok_b, pos_w, pos_b)


def kernel(input_ids, token_table, pos_table, tok_ln_w, tok_ln_b, pos_ln_w, pos_ln_b):
    B, S = input_ids.shape
    V, D = token_table.shape
    ids_flat = input_ids.reshape(B * S).astype(jnp.int32)
    out = _fused_sc(ids_flat, token_table, pos_table, tok_ln_w, tok_ln_b,
                    pos_ln_w, pos_ln_b, B, S)
    return out.reshape(B, S, D)


# speedup vs baseline: 3.0498x; 3.0498x over previous
import functools

import jax
import jax.numpy as jnp
from jax import lax
from jax.experimental import pallas as pl
from jax.experimental.pallas import tpu as pltpu
from jax.experimental.pallas import tpu_sc as plsc


def _sc_gather(ids_flat, table):
    N = ids_flat.shape[0]
    V, D = table.shape
    info = plsc.get_sparse_core_info()
    nw = info.num_cores * info.num_subcores
    rows_per_w = N // nw
    ch = 32
    n_ch = rows_per_w // ch
    mesh = plsc.VectorSubcoreMesh(core_axis_name="c", subcore_axis_name="s")

    @functools.partial(
        pl.kernel,
        mesh=mesh,
        out_type=jax.ShapeDtypeStruct((N, D), jnp.float32),
        scratch_types=[
            pltpu.VMEM((ch,), jnp.int32),
            pltpu.VMEM((ch, D), jnp.float32),
            pltpu.SemaphoreType.DMA,
        ],
    )
    def gather_kernel(ids_hbm, table_hbm, out_hbm, idx_v, rows_v, sem):
        wid = lax.axis_index("s") * info.num_cores + lax.axis_index("c")
        base = wid * rows_per_w

        def body(i, carry):
            off = base + i * ch
            pltpu.sync_copy(ids_hbm.at[pl.ds(off, ch)], idx_v)
            pltpu.async_copy(table_hbm.at[idx_v], rows_v, sem).wait()
            pltpu.sync_copy(rows_v, out_hbm.at[pl.ds(off, ch)])
            return carry

        lax.fori_loop(0, n_ch, body, 0)

    return gather_kernel(ids_flat, table)


def _ln_body(g_ref, p_ref, tw_ref, tb_ref, pw_ref, pb_ref, o_ref):
    x = g_ref[...]
    mu = jnp.mean(x, axis=-1, keepdims=True)
    var = jnp.mean((x - mu) ** 2, axis=-1, keepdims=True)
    tok = (x - mu) * lax.rsqrt(var + 1e-5) * tw_ref[...] + tb_ref[...]
    p = p_ref[...]
    pmu = jnp.mean(p, axis=-1, keepdims=True)
    pvar = jnp.mean((p - pmu) ** 2, axis=-1, keepdims=True)
    pos = (p - pmu) * lax.rsqrt(pvar + 1e-5) * pw_ref[...] + pb_ref[...]
    o_ref[...] = tok + pos


def _tc_ln_chunk(g, pos_table, tw, tb, pw, pb, buf, k, N, B, S, s_chunk):
    D = g.shape[1]
    blk = 512
    sb = s_chunk // blk
    vec = lambda: pl.BlockSpec((1, D), lambda s, b: (0, 0))
    in_specs = [
        pl.BlockSpec((blk, D), lambda s, b: (b * sb + s, 0)),
        pl.BlockSpec((blk, D), lambda s, b: (k * sb + s, 0)),
        vec(), vec(), vec(), vec(),
    ]
    args = [g, pos_table, tw.reshape(1, D), tb.reshape(1, D),
            pw.reshape(1, D), pb.reshape(1, D)]
    kwargs = {}
    if buf is not None:
        in_specs.append(pl.BlockSpec(memory_space=pl.ANY))
        args.append(buf)
        kwargs["input_output_aliases"] = {6: 0}
        body = lambda g_, p_, a_, b_, c_, d_, _buf, o_: _ln_body(
            g_, p_, a_, b_, c_, d_, o_)
    else:
        body = _ln_body
    return pl.pallas_call(
        body,
        grid=(sb, B),
        in_specs=in_specs,
        out_specs=pl.BlockSpec(
            (blk, D), lambda s, b: (b * (S // blk) + k * sb + s, 0)),
        out_shape=jax.ShapeDtypeStruct((N, D), jnp.float32),
        **kwargs,
    )(*args)


def kernel(input_ids, token_table, pos_table, tok_ln_w, tok_ln_b, pos_ln_w, pos_ln_b):
    B, S = input_ids.shape
    V, D = token_table.shape
    n_chunks = 4
    s_chunk = S // n_chunks
    ids32 = input_ids.astype(jnp.int32)
    gs = [
        _sc_gather(ids32[:, k * s_chunk:(k + 1) * s_chunk].reshape(-1),
                   token_table)
        for k in range(n_chunks)
    ]
    buf = None
    for k in range(n_chunks):
        buf = _tc_ln_chunk(gs[k], pos_table, tok_ln_w, tok_ln_b, pos_ln_w,
                           pos_ln_b, buf, k, B * S, B, S, s_chunk)
    return buf.reshape(B, S, D)
